# baseline (speedup 1.0000x reference)
import functools

import jax
import jax.numpy as jnp
from jax import lax
from jax.experimental import pallas as pl
from jax.experimental.pallas import tpu as pltpu
from jax.experimental.pallas import tpu_sc as plsc

N = 10000
E = 320000
D_IN = 128
HID = 16
HEADS = 8
D_OUT = 64

NC = 2
NS = 16
CH = 80
EPT = E // NS
NCHUNK = EPT // CH
NPAD = 10240
ROWS_PER_TILE = NPAD // NS

ROW_BLOCK = 400


def _stage_a_body(x_ref, w_ref, as_ref, ad_ref, f3_ref, adst_ref):
    h = jnp.dot(x_ref[...], w_ref[...], preferred_element_type=jnp.float32)
    asrc = jnp.dot(h, as_ref[...], preferred_element_type=jnp.float32)
    adst = jnp.dot(h, ad_ref[...], preferred_element_type=jnp.float32)
    f3_ref[0] = jnp.concatenate([h[:, 0:64], asrc[:, 0:16]], axis=1)
    f3_ref[1] = jnp.concatenate([h[:, 64:128], asrc[:, 16:32]], axis=1)
    adst_ref[0] = adst[:, 0:16]
    adst_ref[1] = adst[:, 16:32]


def _stage_a(x, W1, As1, Ad1):
    grid = (N // ROW_BLOCK,)
    return pl.pallas_call(
        _stage_a_body,
        grid=grid,
        in_specs=[
            pl.BlockSpec((ROW_BLOCK, D_IN), lambda i: (i, 0)),
            pl.BlockSpec((D_IN, HEADS * HID), lambda i: (0, 0)),
            pl.BlockSpec((HEADS * HID, 32), lambda i: (0, 0)),
            pl.BlockSpec((HEADS * HID, 32), lambda i: (0, 0)),
        ],
        out_specs=[
            pl.BlockSpec((NC, ROW_BLOCK, 80), lambda i: (0, i, 0)),
            pl.BlockSpec((NC, ROW_BLOCK, 16), lambda i: (0, i, 0)),
        ],
        out_shape=[
            jax.ShapeDtypeStruct((NC, NPAD, 80), jnp.float32),
            jax.ShapeDtypeStruct((NC, NPAD, 16), jnp.float32),
        ],
    )(x, W1, As1, Ad1)


def _stage_c_body(accp_ref, b1_ref, r_ref, w2_ref, as2_ref, ad2_ref,
                  f3_ref, adst2_ref):
    a0 = accp_ref[0]
    a1 = accp_ref[1]
    acc = jnp.concatenate([a0[:, 0:64], a1[:, 0:64]], axis=1)
    den = jnp.concatenate([a0[:, 64:80], a1[:, 64:80]], axis=1)
    den_b = jnp.dot(den, r_ref[...], preferred_element_type=jnp.float32)
    h1 = acc / (den_b + 1e-16) + b1_ref[...]
    h1 = jnp.maximum(h1, 0.01 * h1)
    h2 = jnp.dot(h1, w2_ref[...], preferred_element_type=jnp.float32)
    asrc2 = jnp.dot(h2, as2_ref[...], preferred_element_type=jnp.float32)
    adst2 = jnp.dot(h2, ad2_ref[...], preferred_element_type=jnp.float32)
    f3_ref[0] = jnp.concatenate([h2[:, 0:32], asrc2], axis=1)
    f3_ref[1] = jnp.concatenate([h2[:, 32:64], asrc2], axis=1)
    adst2_ref[0] = adst2
    adst2_ref[1] = adst2


def _stage_c(accp, b1r, R, W2, As2, Ad2):
    grid = (N // ROW_BLOCK,)
    return pl.pallas_call(
        _stage_c_body,
        grid=grid,
        in_specs=[
            pl.BlockSpec((NC, ROW_BLOCK, 80), lambda i: (0, i, 0)),
            pl.BlockSpec((1, 128), lambda i: (0, 0)),
            pl.BlockSpec((32, 128), lambda i: (0, 0)),
            pl.BlockSpec((128, D_OUT), lambda i: (0, 0)),
            pl.BlockSpec((D_OUT, 16), lambda i: (0, 0)),
            pl.BlockSpec((D_OUT, 16), lambda i: (0, 0)),
        ],
        out_specs=[
            pl.BlockSpec((NC, ROW_BLOCK, 48), lambda i: (0, i, 0)),
            pl.BlockSpec((NC, ROW_BLOCK, 16), lambda i: (0, i, 0)),
        ],
        out_shape=[
            jax.ShapeDtypeStruct((NC, NPAD, 48), jnp.float32),
            jax.ShapeDtypeStruct((NC, NPAD, 16), jnp.float32),
        ],
    )(accp, b1r, R, W2, As2, Ad2)


def _stage_e_body(accp_ref, b2_ref, r2_ref, out_ref):
    a0 = accp_ref[0]
    a1 = accp_ref[1]
    acc = jnp.concatenate([a0[:, 0:32], a1[:, 0:32]], axis=1)
    den = a0[:, 32:48]
    den_b = jnp.dot(den, r2_ref[...], preferred_element_type=jnp.float32)
    out_ref[...] = acc / (den_b + 1e-16) + b2_ref[...]


def _stage_e(accp, b2r, R2):
    grid = (N // ROW_BLOCK,)
    return pl.pallas_call(
        _stage_e_body,
        grid=grid,
        in_specs=[
            pl.BlockSpec((NC, ROW_BLOCK, 48), lambda i: (0, i, 0)),
            pl.BlockSpec((1, D_OUT), lambda i: (0, 0)),
            pl.BlockSpec((16, D_OUT), lambda i: (0, 0)),
        ],
        out_specs=pl.BlockSpec((ROW_BLOCK, D_OUT), lambda i: (i, 0)),
        out_shape=jax.ShapeDtypeStruct((N, D_OUT), jnp.float32),
    )(accp, b2r, R2)


def _sc_edge_stage(FW, n_heads_sc, c_head):
    ROWW = FW + 16
    n_ch = FW // 16
    mesh = plsc.VectorSubcoreMesh(core_axis_name="c", subcore_axis_name="s")

    @functools.partial(
        pl.kernel,
        mesh=mesh,
        compiler_params=pltpu.CompilerParams(use_tc_tiling_on_sc=False),
        out_type=jax.ShapeDtypeStruct((NC, NPAD, ROWW), jnp.float32),
        scratch_types=[
            pltpu.VMEM((NCHUNK, CH), jnp.int32),
            pltpu.VMEM((NCHUNK, CH), jnp.int32),
            pltpu.VMEM((2, CH, 16), jnp.float32),
            pltpu.VMEM((2, CH, ROWW), jnp.float32),
            pltpu.VMEM((2, CH, ROWW), jnp.float32),
            pltpu.VMEM_SHARED((NPAD, ROWW), jnp.float32),
            pltpu.SemaphoreType.DMA,
            pltpu.SemaphoreType.DMA,
            pltpu.SemaphoreType.DMA,
            pltpu.SemaphoreType.DMA,
        ],
    )
    def k(src_hbm, dst_hbm, adst_hbm, feat_hbm, zrow_hbm,
          accp_hbm,
          sidx_all, didx_all, dbuf, fbuf, obuf, acc_s,
          gsem0, gsem1, ssem0, ssem1):
        gsems = (gsem0, gsem1)
        ssems = (ssem0, ssem1)
        c = lax.axis_index("c")
        s = lax.axis_index("s")
        r0 = s * ROWS_PER_TILE
        pltpu.sync_copy(zrow_hbm.at[pl.ds(r0, ROWS_PER_TILE)],
                        acc_s.at[pl.ds(r0, ROWS_PER_TILE)])
        plsc.subcore_barrier()
        pltpu.sync_copy(src_hbm.at[s], sidx_all)
        pltpu.sync_copy(dst_hbm.at[s], didx_all)

        lane = lax.iota(jnp.int32, 16)

        def fire_gathers(ci, b):
            pltpu.async_copy(adst_hbm.at[c].at[didx_all.at[ci]], dbuf.at[b],
                             gsems[b])
            pltpu.async_copy(feat_hbm.at[c].at[sidx_all.at[ci]], fbuf.at[b],
                             gsems[b])

        def wait_gathers(b):
            pltpu.make_async_copy(adst_hbm.at[c].at[didx_all.at[0]],
                                  dbuf.at[b], gsems[b]).wait()
            pltpu.make_async_copy(feat_hbm.at[c].at[sidx_all.at[0]],
                                  fbuf.at[b], gsems[b]).wait()

        def fire_scatter(ci, b):
            pltpu.async_copy(obuf.at[b], acc_s.at[didx_all.at[ci]],
                             ssems[b], add=True)

        def wait_scatter(b):
            pltpu.make_async_copy(obuf.at[b], acc_s.at[didx_all.at[0]],
                                  ssems[b]).wait()

        def compute(b):
            db = dbuf.at[b]
            fb = fbuf.at[b]
            ob = obuf.at[b]

            @plsc.parallel_loop(0, CH, unroll=8)
            def edge_body(e):
                ee = fb[e, pl.ds(FW, 16)] + db[e, :]
                ee = jnp.maximum(ee, 0.2 * ee)
                w = jnp.exp(ee)
                w = jnp.where(lane < n_heads_sc, w, 0.0)
                ob[e, pl.ds(FW, 16)] = w
                for chi in range(n_ch):
                    wh = w[(chi * 16) // c_head]
                    ob[e, pl.ds(chi * 16, 16)] = (
                        fb[e, pl.ds(chi * 16, 16)] * wh)

        def process(ci, b):
            nb = 1 - b
            fire_gathers(ci + 1, nb)
            wait_gathers(b)

            @pl.when(ci >= 2)
            def _():
                wait_scatter(b)

            compute(b)
            fire_scatter(ci, b)

        fire_gathers(0, 0)

        def super_body(si, carry):
            process(2 * si, 0)
            process(2 * si + 1, 1)
            return carry

        lax.fori_loop(0, NCHUNK // 2 - 1, super_body, 0)
        process(NCHUNK - 2, 0)
        wait_gathers(1)
        wait_scatter(1)
        compute(1)
        fire_scatter(NCHUNK - 1, 1)
        wait_scatter(0)
        wait_scatter(1)

        plsc.subcore_barrier()
        pltpu.sync_copy(acc_s.at[pl.ds(r0, ROWS_PER_TILE)],
                        accp_hbm.at[c, pl.ds(r0, ROWS_PER_TILE)])

    return k


def kernel(x, edge_index, W1, a_src1, a_dst1, b1, W2, a_src2, a_dst2, b2):
    src = edge_index[0].reshape(NS, NCHUNK, CH)
    dst = edge_index[1].reshape(NS, NCHUNK, CH)

    eye8 = jnp.eye(HEADS, dtype=jnp.float32)
    zpad12 = jnp.zeros((HEADS * HID, 12), jnp.float32)
    AsF = (eye8[:, None, :] * a_src1[:, :, None]).reshape(HEADS * HID, HEADS)
    AdF = (eye8[:, None, :] * a_dst1[:, :, None]).reshape(HEADS * HID, HEADS)
    As1 = jnp.concatenate(
        [AsF[:, 0:4], zpad12, AsF[:, 4:8], zpad12], axis=1)
    Ad1 = jnp.concatenate(
        [AdF[:, 0:4], zpad12, AdF[:, 4:8], zpad12], axis=1)
    zpad15 = jnp.zeros((D_OUT, 15), jnp.float32)
    As2 = jnp.concatenate([a_src2.T, zpad15], axis=1)
    Ad2 = jnp.concatenate([a_dst2.T, zpad15], axis=1)
    kr4 = jnp.kron(jnp.eye(4, dtype=jnp.float32),
                   jnp.ones((1, HID), jnp.float32))
    z4_64 = jnp.zeros((4, 64), jnp.float32)
    z12_128 = jnp.zeros((12, HEADS * HID), jnp.float32)
    R1 = jnp.concatenate([
        jnp.concatenate([kr4, z4_64], axis=1),
        z12_128,
        jnp.concatenate([z4_64, kr4], axis=1),
        z12_128,
    ], axis=0)
    R2 = jnp.zeros((16, D_OUT), jnp.float32).at[0].set(1.0)
    b1r = b1.reshape(1, HEADS * HID)
    b2r = b2.reshape(1, D_OUT)

    zrow80 = jnp.zeros((NPAD, 80), jnp.float32)
    zrow48 = jnp.zeros((NPAD, 48), jnp.float32)

    f31, adst1 = _stage_a(x, W1, As1, Ad1)
    accp1 = _sc_edge_stage(64, 4, HID)(
        src, dst, adst1, f31, zrow80)
    f32_, adst2 = _stage_c(accp1, b1r, R1, W2, As2, Ad2)
    accp2 = _sc_edge_stage(32, 1, D_OUT)(
        src, dst, adst2, f32_, zrow48)
    out = _stage_e(accp2, b2r, R2)
    return out


# speedup vs baseline: 1.0099x; 1.0099x over previous
import functools

import jax
import jax.numpy as jnp
from jax import lax
from jax.experimental import pallas as pl
from jax.experimental.pallas import tpu as pltpu
from jax.experimental.pallas import tpu_sc as plsc

N = 10000
E = 320000
D_IN = 128
HID = 16
HEADS = 8
D_OUT = 64

NC = 2
NS = 16
CH = 80
EPT = E // NS
NCHUNK = EPT // CH
NPAD = 10240
ROWS_PER_TILE = NPAD // NS

ROW_BLOCK = 512


def _stage_a_body(x_ref, w_ref, as_ref, ad_ref, f3_ref, adst_ref):
    h = jnp.dot(x_ref[...], w_ref[...], preferred_element_type=jnp.float32)
    asrc = jnp.dot(h, as_ref[...], preferred_element_type=jnp.float32)
    adst = jnp.dot(h, ad_ref[...], preferred_element_type=jnp.float32)
    f3_ref[0] = jnp.concatenate([h[:, 0:64], asrc[:, 0:16]], axis=1)
    f3_ref[1] = jnp.concatenate([h[:, 64:128], asrc[:, 16:32]], axis=1)
    adst_ref[0] = adst[:, 0:16]
    adst_ref[1] = adst[:, 16:32]


def _stage_a(x, W1, As1, Ad1):
    grid = (NPAD // ROW_BLOCK,)
    return pl.pallas_call(
        _stage_a_body,
        grid=grid,
        in_specs=[
            pl.BlockSpec((ROW_BLOCK, D_IN), lambda i: (i, 0)),
            pl.BlockSpec((D_IN, HEADS * HID), lambda i: (0, 0)),
            pl.BlockSpec((HEADS * HID, 32), lambda i: (0, 0)),
            pl.BlockSpec((HEADS * HID, 32), lambda i: (0, 0)),
        ],
        out_specs=[
            pl.BlockSpec((NC, ROW_BLOCK, 80), lambda i: (0, i, 0)),
            pl.BlockSpec((NC, ROW_BLOCK, 16), lambda i: (0, i, 0)),
        ],
        out_shape=[
            jax.ShapeDtypeStruct((NC, NPAD, 80), jnp.float32),
            jax.ShapeDtypeStruct((NC, NPAD, 16), jnp.float32),
        ],
    )(x, W1, As1, Ad1)


def _stage_c_body(accp_ref, b1_ref, r_ref, w2_ref, as2_ref, ad2_ref,
                  f3_ref, adst2_ref):
    a0 = accp_ref[0]
    a1 = accp_ref[1]
    acc = jnp.concatenate([a0[:, 0:64], a1[:, 0:64]], axis=1)
    den = jnp.concatenate([a0[:, 64:80], a1[:, 64:80]], axis=1)
    den_b = jnp.dot(den, r_ref[...], preferred_element_type=jnp.float32)
    h1 = acc / (den_b + 1e-16) + b1_ref[...]
    h1 = jnp.maximum(h1, 0.01 * h1)
    h2 = jnp.dot(h1, w2_ref[...], preferred_element_type=jnp.float32)
    asrc2 = jnp.dot(h2, as2_ref[...], preferred_element_type=jnp.float32)
    adst2 = jnp.dot(h2, ad2_ref[...], preferred_element_type=jnp.float32)
    f3_ref[0] = jnp.concatenate([h2[:, 0:32], asrc2], axis=1)
    f3_ref[1] = jnp.concatenate([h2[:, 32:64], asrc2], axis=1)
    adst2_ref[0] = adst2
    adst2_ref[1] = adst2


def _stage_c(accp, b1r, R, W2, As2, Ad2):
    grid = (NPAD // ROW_BLOCK,)
    return pl.pallas_call(
        _stage_c_body,
        grid=grid,
        in_specs=[
            pl.BlockSpec((NC, ROW_BLOCK, 80), lambda i: (0, i, 0)),
            pl.BlockSpec((1, 128), lambda i: (0, 0)),
            pl.BlockSpec((32, 128), lambda i: (0, 0)),
            pl.BlockSpec((128, D_OUT), lambda i: (0, 0)),
            pl.BlockSpec((D_OUT, 16), lambda i: (0, 0)),
            pl.BlockSpec((D_OUT, 16), lambda i: (0, 0)),
        ],
        out_specs=[
            pl.BlockSpec((NC, ROW_BLOCK, 48), lambda i: (0, i, 0)),
            pl.BlockSpec((NC, ROW_BLOCK, 16), lambda i: (0, i, 0)),
        ],
        out_shape=[
            jax.ShapeDtypeStruct((NC, NPAD, 48), jnp.float32),
            jax.ShapeDtypeStruct((NC, NPAD, 16), jnp.float32),
        ],
    )(accp, b1r, R, W2, As2, Ad2)


def _stage_e_body(accp_ref, b2_ref, r2_ref, out_ref):
    a0 = accp_ref[0]
    a1 = accp_ref[1]
    acc = jnp.concatenate([a0[:, 0:32], a1[:, 0:32]], axis=1)
    den = a0[:, 32:48]
    den_b = jnp.dot(den, r2_ref[...], preferred_element_type=jnp.float32)
    out_ref[...] = acc / (den_b + 1e-16) + b2_ref[...]


def _stage_e(accp, b2r, R2):
    grid = (NPAD // ROW_BLOCK,)
    return pl.pallas_call(
        _stage_e_body,
        grid=grid,
        in_specs=[
            pl.BlockSpec((NC, ROW_BLOCK, 48), lambda i: (0, i, 0)),
            pl.BlockSpec((1, D_OUT), lambda i: (0, 0)),
            pl.BlockSpec((16, D_OUT), lambda i: (0, 0)),
        ],
        out_specs=pl.BlockSpec((ROW_BLOCK, D_OUT), lambda i: (i, 0)),
        out_shape=jax.ShapeDtypeStruct((NPAD, D_OUT), jnp.float32),
    )(accp, b2r, R2)


def _sc_edge_stage(FW, n_heads_sc, c_head):
    ROWW = FW + 16
    n_ch = FW // 16
    mesh = plsc.VectorSubcoreMesh(core_axis_name="c", subcore_axis_name="s")

    @functools.partial(
        pl.kernel,
        mesh=mesh,
        compiler_params=pltpu.CompilerParams(use_tc_tiling_on_sc=False),
        out_type=jax.ShapeDtypeStruct((NC, NPAD, ROWW), jnp.float32),
        scratch_types=[
            pltpu.VMEM((NCHUNK, CH), jnp.int32),
            pltpu.VMEM((NCHUNK, CH), jnp.int32),
            pltpu.VMEM((2, CH, 16), jnp.float32),
            pltpu.VMEM((2, CH, ROWW), jnp.float32),
            pltpu.VMEM((2, CH, ROWW), jnp.float32),
            pltpu.VMEM_SHARED((NPAD, ROWW), jnp.float32),
            pltpu.SemaphoreType.DMA,
            pltpu.SemaphoreType.DMA,
            pltpu.SemaphoreType.DMA,
            pltpu.SemaphoreType.DMA,
        ],
    )
    def k(src_hbm, dst_hbm, adst_hbm, feat_hbm, zrow_hbm,
          accp_hbm,
          sidx_all, didx_all, dbuf, fbuf, obuf, acc_s,
          gsem0, gsem1, ssem0, ssem1):
        gsems = (gsem0, gsem1)
        ssems = (ssem0, ssem1)
        c = lax.axis_index("c")
        s = lax.axis_index("s")
        r0 = s * ROWS_PER_TILE
        pltpu.sync_copy(zrow_hbm.at[pl.ds(r0, ROWS_PER_TILE)],
                        acc_s.at[pl.ds(r0, ROWS_PER_TILE)])
        plsc.subcore_barrier()
        pltpu.sync_copy(src_hbm.at[s], sidx_all)
        pltpu.sync_copy(dst_hbm.at[s], didx_all)

        lane = lax.iota(jnp.int32, 16)

        def fire_gathers(ci, b):
            pltpu.async_copy(adst_hbm.at[c].at[didx_all.at[ci]], dbuf.at[b],
                             gsems[b])
            pltpu.async_copy(feat_hbm.at[c].at[sidx_all.at[ci]], fbuf.at[b],
                             gsems[b])

        def wait_gathers(b):
            pltpu.make_async_copy(adst_hbm.at[c].at[didx_all.at[0]],
                                  dbuf.at[b], gsems[b]).wait()
            pltpu.make_async_copy(feat_hbm.at[c].at[sidx_all.at[0]],
                                  fbuf.at[b], gsems[b]).wait()

        def fire_scatter(ci, b):
            pltpu.async_copy(obuf.at[b], acc_s.at[didx_all.at[ci]],
                             ssems[b], add=True)

        def wait_scatter(b):
            pltpu.make_async_copy(obuf.at[b], acc_s.at[didx_all.at[0]],
                                  ssems[b]).wait()

        def compute(b):
            db = dbuf.at[b]
            fb = fbuf.at[b]
            ob = obuf.at[b]

            @plsc.parallel_loop(0, CH, unroll=8)
            def edge_body(e):
                ee = fb[e, pl.ds(FW, 16)] + db[e, :]
                ee = jnp.maximum(ee, 0.2 * ee)
                w = jnp.exp(ee)
                w = jnp.where(lane < n_heads_sc, w, 0.0)
                ob[e, pl.ds(FW, 16)] = w
                for chi in range(n_ch):
                    wh = w[(chi * 16) // c_head]
                    ob[e, pl.ds(chi * 16, 16)] = (
                        fb[e, pl.ds(chi * 16, 16)] * wh)

        def process(ci, b):
            nb = 1 - b
            fire_gathers(ci + 1, nb)
            wait_gathers(b)

            @pl.when(ci >= 2)
            def _():
                wait_scatter(b)

            compute(b)
            fire_scatter(ci, b)

        fire_gathers(0, 0)

        def super_body(si, carry):
            process(2 * si, 0)
            process(2 * si + 1, 1)
            return carry

        lax.fori_loop(0, NCHUNK // 2 - 1, super_body, 0)
        process(NCHUNK - 2, 0)
        wait_gathers(1)
        wait_scatter(1)
        compute(1)
        fire_scatter(NCHUNK - 1, 1)
        wait_scatter(0)
        wait_scatter(1)

        plsc.subcore_barrier()
        pltpu.sync_copy(acc_s.at[pl.ds(r0, ROWS_PER_TILE)],
                        accp_hbm.at[c, pl.ds(r0, ROWS_PER_TILE)])

    return k


def kernel(x, edge_index, W1, a_src1, a_dst1, b1, W2, a_src2, a_dst2, b2):
    src = edge_index[0].reshape(NS, NCHUNK, CH)
    dst = edge_index[1].reshape(NS, NCHUNK, CH)

    eye8 = jnp.eye(HEADS, dtype=jnp.float32)
    zpad12 = jnp.zeros((HEADS * HID, 12), jnp.float32)
    AsF = (eye8[:, None, :] * a_src1[:, :, None]).reshape(HEADS * HID, HEADS)
    AdF = (eye8[:, None, :] * a_dst1[:, :, None]).reshape(HEADS * HID, HEADS)
    As1 = jnp.concatenate(
        [AsF[:, 0:4], zpad12, AsF[:, 4:8], zpad12], axis=1)
    Ad1 = jnp.concatenate(
        [AdF[:, 0:4], zpad12, AdF[:, 4:8], zpad12], axis=1)
    zpad15 = jnp.zeros((D_OUT, 15), jnp.float32)
    As2 = jnp.concatenate([a_src2.T, zpad15], axis=1)
    Ad2 = jnp.concatenate([a_dst2.T, zpad15], axis=1)
    kr4 = jnp.kron(jnp.eye(4, dtype=jnp.float32),
                   jnp.ones((1, HID), jnp.float32))
    z4_64 = jnp.zeros((4, 64), jnp.float32)
    z12_128 = jnp.zeros((12, HEADS * HID), jnp.float32)
    R1 = jnp.concatenate([
        jnp.concatenate([kr4, z4_64], axis=1),
        z12_128,
        jnp.concatenate([z4_64, kr4], axis=1),
        z12_128,
    ], axis=0)
    R2 = jnp.zeros((16, D_OUT), jnp.float32).at[0].set(1.0)
    b1r = b1.reshape(1, HEADS * HID)
    b2r = b2.reshape(1, D_OUT)

    zrow80 = jnp.zeros((NPAD, 80), jnp.float32)
    zrow48 = jnp.zeros((NPAD, 48), jnp.float32)

    xp = jnp.pad(x, ((0, NPAD - N), (0, 0)))
    f31, adst1 = _stage_a(xp, W1, As1, Ad1)
    accp1 = _sc_edge_stage(64, 4, HID)(
        src, dst, adst1, f31, zrow80)
    f32_, adst2 = _stage_c(accp1, b1r, R1, W2, As2, Ad2)
    accp2 = _sc_edge_stage(32, 1, D_OUT)(
        src, dst, adst2, f32_, zrow48)
    out = _stage_e(accp2, b2r, R2)
    return out[:N]


# speedup vs baseline: 1.0451x; 1.0349x over previous
import functools

import jax
import jax.numpy as jnp
from jax import lax
from jax.experimental import pallas as pl
from jax.experimental.pallas import tpu as pltpu
from jax.experimental.pallas import tpu_sc as plsc

N = 10000
E = 320000
D_IN = 128
HID = 16
HEADS = 8
D_OUT = 64

NC = 2
NS = 16
CH = 80
EPT = E // NS
NCHUNK = EPT // CH
NPAD = 10240
ROWS_PER_TILE = NPAD // NS

ROW_BLOCK = 512


def _stage_a_body(x_ref, w_ref, as_ref, ad_ref, f3_ref, adst_ref):
    h = jnp.dot(x_ref[...], w_ref[...], preferred_element_type=jnp.float32)
    asrc = jnp.dot(h, as_ref[...], preferred_element_type=jnp.float32)
    adst = jnp.dot(h, ad_ref[...], preferred_element_type=jnp.float32)
    f3_ref[0] = jnp.concatenate([h[:, 0:64], asrc[:, 0:16]], axis=1)
    f3_ref[1] = jnp.concatenate([h[:, 64:128], asrc[:, 16:32]], axis=1)
    adst_ref[0] = adst[:, 0:16]
    adst_ref[1] = adst[:, 16:32]


def _stage_a(x, W1, As1, Ad1):
    grid = (NPAD // ROW_BLOCK,)
    return pl.pallas_call(
        _stage_a_body,
        grid=grid,
        in_specs=[
            pl.BlockSpec((ROW_BLOCK, D_IN), lambda i: (i, 0)),
            pl.BlockSpec((D_IN, HEADS * HID), lambda i: (0, 0)),
            pl.BlockSpec((HEADS * HID, 32), lambda i: (0, 0)),
            pl.BlockSpec((HEADS * HID, 32), lambda i: (0, 0)),
        ],
        out_specs=[
            pl.BlockSpec((NC, ROW_BLOCK, 80), lambda i: (0, i, 0)),
            pl.BlockSpec((NC, ROW_BLOCK, 16), lambda i: (0, i, 0)),
        ],
        out_shape=[
            jax.ShapeDtypeStruct((NC, NPAD, 80), jnp.float32),
            jax.ShapeDtypeStruct((NC, NPAD, 16), jnp.float32),
        ],
    )(x, W1, As1, Ad1)


def _stage_c_body(accp_ref, b1_ref, r_ref, w2_ref, as2_ref, ad2_ref,
                  f3_ref, adst2_ref):
    a0 = accp_ref[0]
    a1 = accp_ref[1]
    acc = jnp.concatenate([a0[:, 0:64], a1[:, 0:64]], axis=1)
    den = jnp.concatenate([a0[:, 64:80], a1[:, 64:80]], axis=1)
    den_b = jnp.dot(den, r_ref[...], preferred_element_type=jnp.float32)
    h1 = acc / (den_b + 1e-16) + b1_ref[...]
    h1 = jnp.maximum(h1, 0.01 * h1)
    h2 = jnp.dot(h1, w2_ref[...], preferred_element_type=jnp.float32)
    asrc2 = jnp.dot(h2, as2_ref[...], preferred_element_type=jnp.float32)
    adst2 = jnp.dot(h2, ad2_ref[...], preferred_element_type=jnp.float32)
    f3_ref[0] = jnp.concatenate([h2[:, 0:32], asrc2], axis=1)
    f3_ref[1] = jnp.concatenate([h2[:, 32:64], asrc2], axis=1)
    adst2_ref[0] = adst2
    adst2_ref[1] = adst2


def _stage_c(accp, b1r, R, W2, As2, Ad2):
    grid = (NPAD // ROW_BLOCK,)
    return pl.pallas_call(
        _stage_c_body,
        grid=grid,
        in_specs=[
            pl.BlockSpec((NC, ROW_BLOCK, 80), lambda i: (0, i, 0)),
            pl.BlockSpec((1, 128), lambda i: (0, 0)),
            pl.BlockSpec((32, 128), lambda i: (0, 0)),
            pl.BlockSpec((128, D_OUT), lambda i: (0, 0)),
            pl.BlockSpec((D_OUT, 16), lambda i: (0, 0)),
            pl.BlockSpec((D_OUT, 16), lambda i: (0, 0)),
        ],
        out_specs=[
            pl.BlockSpec((NC, ROW_BLOCK, 48), lambda i: (0, i, 0)),
            pl.BlockSpec((NC, ROW_BLOCK, 16), lambda i: (0, i, 0)),
        ],
        out_shape=[
            jax.ShapeDtypeStruct((NC, NPAD, 48), jnp.float32),
            jax.ShapeDtypeStruct((NC, NPAD, 16), jnp.float32),
        ],
    )(accp, b1r, R, W2, As2, Ad2)


def _stage_e_body(accp_ref, b2_ref, r2_ref, out_ref):
    a0 = accp_ref[0]
    a1 = accp_ref[1]
    acc = jnp.concatenate([a0[:, 0:32], a1[:, 0:32]], axis=1)
    den = a0[:, 32:48]
    den_b = jnp.dot(den, r2_ref[...], preferred_element_type=jnp.float32)
    out_ref[...] = acc / (den_b + 1e-16) + b2_ref[...]


def _stage_e(accp, b2r, R2):
    grid = (NPAD // ROW_BLOCK,)
    return pl.pallas_call(
        _stage_e_body,
        grid=grid,
        in_specs=[
            pl.BlockSpec((NC, ROW_BLOCK, 48), lambda i: (0, i, 0)),
            pl.BlockSpec((1, D_OUT), lambda i: (0, 0)),
            pl.BlockSpec((16, D_OUT), lambda i: (0, 0)),
        ],
        out_specs=pl.BlockSpec((ROW_BLOCK, D_OUT), lambda i: (i, 0)),
        out_shape=jax.ShapeDtypeStruct((NPAD, D_OUT), jnp.float32),
    )(accp, b2r, R2)


def _sc_edge_stage(FW, n_heads_sc, c_head, finalize=False):
    ROWW = FW + 16
    n_ch = FW // 16
    mesh = plsc.VectorSubcoreMesh(core_axis_name="c", subcore_axis_name="s")

    if finalize:
        out_type = jax.ShapeDtypeStruct((NPAD, NC * FW), jnp.float32)
        extra_scratch = [
            pltpu.VMEM((CH, FW), jnp.float32),
            pltpu.VMEM((NC * FW,), jnp.float32),
        ]
    else:
        out_type = jax.ShapeDtypeStruct((NC, NPAD, ROWW), jnp.float32)
        extra_scratch = []

    @functools.partial(
        pl.kernel,
        mesh=mesh,
        compiler_params=pltpu.CompilerParams(use_tc_tiling_on_sc=False),
        out_type=out_type,
        scratch_types=[
            pltpu.VMEM((NCHUNK, CH), jnp.int32),
            pltpu.VMEM((NCHUNK, CH), jnp.int32),
            pltpu.VMEM((2, CH, 16), jnp.float32),
            pltpu.VMEM((2, CH, ROWW), jnp.float32),
            pltpu.VMEM((2, CH, ROWW), jnp.float32),
        ] + extra_scratch + [
            pltpu.VMEM_SHARED((NPAD, ROWW), jnp.float32),
            pltpu.SemaphoreType.DMA,
            pltpu.SemaphoreType.DMA,
            pltpu.SemaphoreType.DMA,
            pltpu.SemaphoreType.DMA,
        ],
    )
    def k(src_hbm, dst_hbm, adst_hbm, feat_hbm, zrow_hbm, *rest):
        if finalize:
            (b2_hbm, out_hbm, sidx_all, didx_all, dbuf, fbuf, obuf,
             ob2, b2v, acc_s, gsem0, gsem1, ssem0, ssem1) = rest
        else:
            (accp_hbm, sidx_all, didx_all, dbuf, fbuf, obuf,
             acc_s, gsem0, gsem1, ssem0, ssem1) = rest
        gsems = (gsem0, gsem1)
        ssems = (ssem0, ssem1)
        c = lax.axis_index("c")
        s = lax.axis_index("s")
        r0 = s * ROWS_PER_TILE
        pltpu.sync_copy(zrow_hbm.at[pl.ds(r0, ROWS_PER_TILE)],
                        acc_s.at[pl.ds(r0, ROWS_PER_TILE)])
        plsc.subcore_barrier()
        pltpu.sync_copy(src_hbm.at[s], sidx_all)
        pltpu.sync_copy(dst_hbm.at[s], didx_all)

        lane = lax.iota(jnp.int32, 16)

        def fire_gathers(ci, b):
            pltpu.async_copy(adst_hbm.at[c].at[didx_all.at[ci]], dbuf.at[b],
                             gsems[b])
            pltpu.async_copy(feat_hbm.at[c].at[sidx_all.at[ci]], fbuf.at[b],
                             gsems[b])

        def wait_gathers(b):
            pltpu.make_async_copy(adst_hbm.at[c].at[didx_all.at[0]],
                                  dbuf.at[b], gsems[b]).wait()
            pltpu.make_async_copy(feat_hbm.at[c].at[sidx_all.at[0]],
                                  fbuf.at[b], gsems[b]).wait()

        def fire_scatter(ci, b):
            pltpu.async_copy(obuf.at[b], acc_s.at[didx_all.at[ci]],
                             ssems[b], add=True)

        def wait_scatter(b):
            pltpu.make_async_copy(obuf.at[b], acc_s.at[didx_all.at[0]],
                                  ssems[b]).wait()

        def compute(b):
            db = dbuf.at[b]
            fb = fbuf.at[b]
            ob = obuf.at[b]

            @plsc.parallel_loop(0, CH, unroll=8)
            def edge_body(e):
                ee = fb[e, pl.ds(FW, 16)] + db[e, :]
                ee = jnp.maximum(ee, 0.2 * ee)
                w = jnp.exp(ee)
                w = jnp.where(lane < n_heads_sc, w, 0.0)
                ob[e, pl.ds(FW, 16)] = w
                for chi in range(n_ch):
                    wh = w[(chi * 16) // c_head]
                    ob[e, pl.ds(chi * 16, 16)] = (
                        fb[e, pl.ds(chi * 16, 16)] * wh)

        def process(ci, b):
            nb = 1 - b
            fire_gathers(ci + 1, nb)
            wait_gathers(b)

            @pl.when(ci >= 2)
            def _():
                wait_scatter(b)

            compute(b)
            fire_scatter(ci, b)

        fire_gathers(0, 0)

        def super_body(si, carry):
            process(2 * si, 0)
            process(2 * si + 1, 1)
            return carry

        lax.fori_loop(0, NCHUNK // 2 - 1, super_body, 0)
        process(NCHUNK - 2, 0)
        wait_gathers(1)
        wait_scatter(1)
        compute(1)
        fire_scatter(NCHUNK - 1, 1)
        wait_scatter(0)
        wait_scatter(1)

        plsc.subcore_barrier()
        if not finalize:
            pltpu.sync_copy(acc_s.at[pl.ds(r0, ROWS_PER_TILE)],
                            accp_hbm.at[c, pl.ds(r0, ROWS_PER_TILE)])
        else:
            pltpu.sync_copy(b2_hbm, b2v)
            b2parts = [b2v[pl.ds(c * FW + 16 * g, 16)]
                       for g in range(n_ch)]
            for kk in range(ROWS_PER_TILE // CH):
                rs = r0 + kk * CH
                pltpu.sync_copy(acc_s.at[pl.ds(rs, CH)], fbuf.at[0])
                fb0 = fbuf.at[0]

                @plsc.parallel_loop(0, CH, unroll=8)
                def row_body(r):
                    d = fb0[r, pl.ds(FW, 16)]
                    den = d[0] + 1e-16
                    for g in range(n_ch):
                        ob2[r, pl.ds(16 * g, 16)] = (
                            fb0[r, pl.ds(16 * g, 16)] / den + b2parts[g])

                pltpu.sync_copy(ob2,
                                out_hbm.at[pl.ds(rs, CH), pl.ds(c * FW, FW)])

    return k


def kernel(x, edge_index, W1, a_src1, a_dst1, b1, W2, a_src2, a_dst2, b2):
    src = edge_index[0].reshape(NS, NCHUNK, CH)
    dst = edge_index[1].reshape(NS, NCHUNK, CH)

    eye8 = jnp.eye(HEADS, dtype=jnp.float32)
    zpad12 = jnp.zeros((HEADS * HID, 12), jnp.float32)
    AsF = (eye8[:, None, :] * a_src1[:, :, None]).reshape(HEADS * HID, HEADS)
    AdF = (eye8[:, None, :] * a_dst1[:, :, None]).reshape(HEADS * HID, HEADS)
    As1 = jnp.concatenate(
        [AsF[:, 0:4], zpad12, AsF[:, 4:8], zpad12], axis=1)
    Ad1 = jnp.concatenate(
        [AdF[:, 0:4], zpad12, AdF[:, 4:8], zpad12], axis=1)
    zpad15 = jnp.zeros((D_OUT, 15), jnp.float32)
    As2 = jnp.concatenate([a_src2.T, zpad15], axis=1)
    Ad2 = jnp.concatenate([a_dst2.T, zpad15], axis=1)
    kr4 = jnp.kron(jnp.eye(4, dtype=jnp.float32),
                   jnp.ones((1, HID), jnp.float32))
    z4_64 = jnp.zeros((4, 64), jnp.float32)
    z12_128 = jnp.zeros((12, HEADS * HID), jnp.float32)
    R1 = jnp.concatenate([
        jnp.concatenate([kr4, z4_64], axis=1),
        z12_128,
        jnp.concatenate([z4_64, kr4], axis=1),
        z12_128,
    ], axis=0)
    R2 = jnp.zeros((16, D_OUT), jnp.float32).at[0].set(1.0)
    b1r = b1.reshape(1, HEADS * HID)
    b2r = b2.reshape(1, D_OUT)

    zrow80 = jnp.zeros((NPAD, 80), jnp.float32)
    zrow48 = jnp.zeros((NPAD, 48), jnp.float32)

    xp = jnp.pad(x, ((0, NPAD - N), (0, 0)))
    f31, adst1 = _stage_a(xp, W1, As1, Ad1)
    accp1 = _sc_edge_stage(64, 4, HID)(
        src, dst, adst1, f31, zrow80)
    f32_, adst2 = _stage_c(accp1, b1r, R1, W2, As2, Ad2)
    out = _sc_edge_stage(32, 1, D_OUT, finalize=True)(
        src, dst, adst2, f32_, zrow48, b2)
    return out[:N]


# speedup vs baseline: 1.0596x; 1.0138x over previous
import functools

import jax
import jax.numpy as jnp
from jax import lax
from jax.experimental import pallas as pl
from jax.experimental.pallas import tpu as pltpu
from jax.experimental.pallas import tpu_sc as plsc

N = 10000
E = 320000
D_IN = 128
HID = 16
HEADS = 8
D_OUT = 64

NC = 2
NS = 16
CH = 80
EPT = E // NS
NCHUNK = EPT // CH
NPAD = 10240
ROWS_PER_TILE = NPAD // NS

ROW_BLOCK = 512


def _stage_a_body(x_ref, w_ref, as_ref, ad_ref, f3_ref, adst_ref):
    h = jnp.dot(x_ref[...], w_ref[...], preferred_element_type=jnp.float32)
    asrc = jnp.dot(h, as_ref[...], preferred_element_type=jnp.float32)
    adst = jnp.dot(h, ad_ref[...], preferred_element_type=jnp.float32)
    f3_ref[0] = jnp.concatenate([h[:, 0:64], asrc[:, 0:16]], axis=1)
    f3_ref[1] = jnp.concatenate([h[:, 64:128], asrc[:, 16:32]], axis=1)
    adst_ref[0] = adst[:, 0:16]
    adst_ref[1] = adst[:, 16:32]


def _stage_a(x, W1, As1, Ad1):
    grid = (NPAD // ROW_BLOCK,)
    return pl.pallas_call(
        _stage_a_body,
        grid=grid,
        in_specs=[
            pl.BlockSpec((ROW_BLOCK, D_IN), lambda i: (i, 0)),
            pl.BlockSpec((D_IN, HEADS * HID), lambda i: (0, 0)),
            pl.BlockSpec((HEADS * HID, 32), lambda i: (0, 0)),
            pl.BlockSpec((HEADS * HID, 32), lambda i: (0, 0)),
        ],
        out_specs=[
            pl.BlockSpec((NC, ROW_BLOCK, 80), lambda i: (0, i, 0)),
            pl.BlockSpec((NC, ROW_BLOCK, 16), lambda i: (0, i, 0)),
        ],
        out_shape=[
            jax.ShapeDtypeStruct((NC, NPAD, 80), jnp.float32),
            jax.ShapeDtypeStruct((NC, NPAD, 16), jnp.float32),
        ],
    )(x, W1, As1, Ad1)


def _stage_c_body(h1_ref, w2_ref, as2_ref, ad2_ref, f3_ref, adst2_ref):
    h1 = h1_ref[...]
    h2 = jnp.dot(h1, w2_ref[...], preferred_element_type=jnp.float32)
    asrc2 = jnp.dot(h2, as2_ref[...], preferred_element_type=jnp.float32)
    adst2 = jnp.dot(h2, ad2_ref[...], preferred_element_type=jnp.float32)
    f3_ref[0] = jnp.concatenate([h2[:, 0:32], asrc2], axis=1)
    f3_ref[1] = jnp.concatenate([h2[:, 32:64], asrc2], axis=1)
    adst2_ref[0] = adst2
    adst2_ref[1] = adst2


def _stage_c(h1, W2, As2, Ad2):
    grid = (NPAD // ROW_BLOCK,)
    return pl.pallas_call(
        _stage_c_body,
        grid=grid,
        in_specs=[
            pl.BlockSpec((ROW_BLOCK, 128), lambda i: (i, 0)),
            pl.BlockSpec((128, D_OUT), lambda i: (0, 0)),
            pl.BlockSpec((D_OUT, 16), lambda i: (0, 0)),
            pl.BlockSpec((D_OUT, 16), lambda i: (0, 0)),
        ],
        out_specs=[
            pl.BlockSpec((NC, ROW_BLOCK, 48), lambda i: (0, i, 0)),
            pl.BlockSpec((NC, ROW_BLOCK, 16), lambda i: (0, i, 0)),
        ],
        out_shape=[
            jax.ShapeDtypeStruct((NC, NPAD, 48), jnp.float32),
            jax.ShapeDtypeStruct((NC, NPAD, 16), jnp.float32),
        ],
    )(h1, W2, As2, Ad2)


def _sc_edge_stage(FW, n_heads_sc, c_head, finalize=False, slope=None):
    ROWW = FW + 16
    n_ch = FW // 16
    mesh = plsc.VectorSubcoreMesh(core_axis_name="c", subcore_axis_name="s")

    if finalize:
        out_type = jax.ShapeDtypeStruct((NPAD, NC * FW), jnp.float32)
        extra_scratch = [
            pltpu.VMEM((CH, FW), jnp.float32),
            pltpu.VMEM((NC * FW,), jnp.float32),
        ]
    else:
        out_type = jax.ShapeDtypeStruct((NC, NPAD, ROWW), jnp.float32)
        extra_scratch = []

    @functools.partial(
        pl.kernel,
        mesh=mesh,
        compiler_params=pltpu.CompilerParams(use_tc_tiling_on_sc=False),
        out_type=out_type,
        scratch_types=[
            pltpu.VMEM((NCHUNK, CH), jnp.int32),
            pltpu.VMEM((NCHUNK, CH), jnp.int32),
            pltpu.VMEM((2, CH, 16), jnp.float32),
            pltpu.VMEM((2, CH, ROWW), jnp.float32),
            pltpu.VMEM((2, CH, ROWW), jnp.float32),
        ] + extra_scratch + [
            pltpu.VMEM_SHARED((NPAD, ROWW), jnp.float32),
            pltpu.SemaphoreType.DMA,
            pltpu.SemaphoreType.DMA,
            pltpu.SemaphoreType.DMA,
            pltpu.SemaphoreType.DMA,
        ],
    )
    def k(src_hbm, dst_hbm, adst_hbm, feat_hbm, zrow_hbm, *rest):
        if finalize:
            (b2_hbm, out_hbm, sidx_all, didx_all, dbuf, fbuf, obuf,
             ob2, b2v, acc_s, gsem0, gsem1, ssem0, ssem1) = rest
        else:
            (accp_hbm, sidx_all, didx_all, dbuf, fbuf, obuf,
             acc_s, gsem0, gsem1, ssem0, ssem1) = rest
        gsems = (gsem0, gsem1)
        ssems = (ssem0, ssem1)
        c = lax.axis_index("c")
        s = lax.axis_index("s")
        r0 = s * ROWS_PER_TILE
        pltpu.sync_copy(zrow_hbm.at[pl.ds(r0, ROWS_PER_TILE)],
                        acc_s.at[pl.ds(r0, ROWS_PER_TILE)])
        plsc.subcore_barrier()
        pltpu.sync_copy(src_hbm.at[s], sidx_all)
        pltpu.sync_copy(dst_hbm.at[s], didx_all)

        lane = lax.iota(jnp.int32, 16)

        def fire_gathers(ci, b):
            pltpu.async_copy(adst_hbm.at[c].at[didx_all.at[ci]], dbuf.at[b],
                             gsems[b])
            pltpu.async_copy(feat_hbm.at[c].at[sidx_all.at[ci]], fbuf.at[b],
                             gsems[b])

        def wait_gathers(b):
            pltpu.make_async_copy(adst_hbm.at[c].at[didx_all.at[0]],
                                  dbuf.at[b], gsems[b]).wait()
            pltpu.make_async_copy(feat_hbm.at[c].at[sidx_all.at[0]],
                                  fbuf.at[b], gsems[b]).wait()

        def fire_scatter(ci, b):
            pltpu.async_copy(obuf.at[b], acc_s.at[didx_all.at[ci]],
                             ssems[b], add=True)

        def wait_scatter(b):
            pltpu.make_async_copy(obuf.at[b], acc_s.at[didx_all.at[0]],
                                  ssems[b]).wait()

        def compute(b):
            db = dbuf.at[b]
            fb = fbuf.at[b]
            ob = obuf.at[b]

            @plsc.parallel_loop(0, CH, unroll=8)
            def edge_body(e):
                ee = fb[e, pl.ds(FW, 16)] + db[e, :]
                ee = jnp.maximum(ee, 0.2 * ee)
                w = jnp.exp(ee)
                w = jnp.where(lane < n_heads_sc, w, 0.0)
                ob[e, pl.ds(FW, 16)] = w
                for chi in range(n_ch):
                    wh = w[(chi * 16) // c_head]
                    ob[e, pl.ds(chi * 16, 16)] = (
                        fb[e, pl.ds(chi * 16, 16)] * wh)

        def process(ci, b):
            nb = 1 - b
            fire_gathers(ci + 1, nb)
            wait_gathers(b)

            @pl.when(ci >= 2)
            def _():
                wait_scatter(b)

            compute(b)
            fire_scatter(ci, b)

        fire_gathers(0, 0)

        def super_body(si, carry):
            process(2 * si, 0)
            process(2 * si + 1, 1)
            return carry

        lax.fori_loop(0, NCHUNK // 2 - 1, super_body, 0)
        process(NCHUNK - 2, 0)
        wait_gathers(1)
        wait_scatter(1)
        compute(1)
        fire_scatter(NCHUNK - 1, 1)
        wait_scatter(0)
        wait_scatter(1)

        plsc.subcore_barrier()
        if not finalize:
            pltpu.sync_copy(acc_s.at[pl.ds(r0, ROWS_PER_TILE)],
                            accp_hbm.at[c, pl.ds(r0, ROWS_PER_TILE)])
        else:
            pltpu.sync_copy(b2_hbm, b2v)
            b2parts = [b2v[pl.ds(c * FW + 16 * g, 16)]
                       for g in range(n_ch)]
            for kk in range(ROWS_PER_TILE // CH):
                rs = r0 + kk * CH
                pltpu.sync_copy(acc_s.at[pl.ds(rs, CH)], fbuf.at[0])
                fb0 = fbuf.at[0]

                @plsc.parallel_loop(0, CH, unroll=8)
                def row_body(r):
                    d = fb0[r, pl.ds(FW, 16)]
                    for g in range(n_ch):
                        den = d[(16 * g) // c_head] + 1e-16
                        v = fb0[r, pl.ds(16 * g, 16)] / den + b2parts[g]
                        if slope is not None:
                            v = jnp.maximum(v, slope * v)
                        ob2[r, pl.ds(16 * g, 16)] = v

                pltpu.sync_copy(ob2,
                                out_hbm.at[pl.ds(rs, CH), pl.ds(c * FW, FW)])

    return k


def kernel(x, edge_index, W1, a_src1, a_dst1, b1, W2, a_src2, a_dst2, b2):
    src = edge_index[0].reshape(NS, NCHUNK, CH)
    dst = edge_index[1].reshape(NS, NCHUNK, CH)

    eye8 = jnp.eye(HEADS, dtype=jnp.float32)
    zpad12 = jnp.zeros((HEADS * HID, 12), jnp.float32)
    AsF = (eye8[:, None, :] * a_src1[:, :, None]).reshape(HEADS * HID, HEADS)
    AdF = (eye8[:, None, :] * a_dst1[:, :, None]).reshape(HEADS * HID, HEADS)
    As1 = jnp.concatenate(
        [AsF[:, 0:4], zpad12, AsF[:, 4:8], zpad12], axis=1)
    Ad1 = jnp.concatenate(
        [AdF[:, 0:4], zpad12, AdF[:, 4:8], zpad12], axis=1)
    zpad15 = jnp.zeros((D_OUT, 15), jnp.float32)
    As2 = jnp.concatenate([a_src2.T, zpad15], axis=1)
    Ad2 = jnp.concatenate([a_dst2.T, zpad15], axis=1)
    zrow80 = jnp.zeros((NPAD, 80), jnp.float32)
    zrow48 = jnp.zeros((NPAD, 48), jnp.float32)

    xp = jnp.pad(x, ((0, NPAD - N), (0, 0)))
    f31, adst1 = _stage_a(xp, W1, As1, Ad1)
    h1 = _sc_edge_stage(64, 4, HID, finalize=True, slope=0.01)(
        src, dst, adst1, f31, zrow80, b1)
    f32_, adst2 = _stage_c(h1, W2, As2, Ad2)
    out = _sc_edge_stage(32, 1, D_OUT, finalize=True)(
        src, dst, adst2, f32_, zrow48, b2)
    return out[:N]
